# baseline (speedup 1.0000x reference)
import functools

import jax
import jax.numpy as jnp
from jax import lax
from jax.experimental import pallas as pl
from jax.experimental.pallas import tpu as pltpu
from jax.experimental.pallas import tpu_sc as plsc


def _sc_gather(B, H, K, dtype, chunk=128, wave=8):
    info = plsc.get_sparse_core_info()
    NC, NS = info.num_cores, info.num_subcores
    NW = NC * NS
    bw = B // NW
    hw = bw * H

    mesh = plsc.VectorSubcoreMesh(core_axis_name="c", subcore_axis_name="s")

    @functools.partial(
        pl.kernel,
        mesh=mesh,
        out_type=[
            jax.ShapeDtypeStruct((B * H, K), dtype),
            jax.ShapeDtypeStruct((B, K), dtype),
            jax.ShapeDtypeStruct((B, K), dtype),
        ],
        scratch_types=[
            pltpu.VMEM((bw,), jnp.int32),
            pltpu.VMEM((bw,), jnp.int32),
            pltpu.VMEM((hw,), jnp.int32),
            pltpu.VMEM((hw,), jnp.int32),
            pltpu.VMEM((bw, K), dtype),
            pltpu.VMEM((bw, K), dtype),
            pltpu.VMEM((hw, K), dtype),
            pltpu.SemaphoreType.DMA,
        ],
        compiler_params=pltpu.CompilerParams(use_tc_tiling_on_sc=False),
    )
    def kern(users_r, items_r, hpos_r, ui_r, gu_tab, gi_tab,
             gih_out, gu_out, gi_out,
             idx_u, idx_i, pos_v, hist_v, gu_v, gi_v, rows_v, sem):
        wid = lax.axis_index("s") * NC + lax.axis_index("c")
        base = wid * bw
        hb = wid * hw
        pltpu.sync_copy(users_r.at[pl.ds(base, bw)], idx_u)
        pltpu.sync_copy(items_r.at[pl.ds(base, bw)], idx_i)
        pltpu.sync_copy(hpos_r.at[pl.ds(hb, hw)], pos_v)
        cps = [pltpu.async_copy(gu_tab.at[idx_u], gu_v, sem),
               pltpu.async_copy(gi_tab.at[idx_i], gi_v, sem)]
        for g in range(hw // chunk):
            cps.append(pltpu.async_copy(
                ui_r.at[pos_v.at[pl.ds(g * chunk, chunk)]],
                hist_v.at[pl.ds(g * chunk, chunk)], sem))
            if len(cps) >= wave:
                for cp in cps:
                    cp.wait()
                cps = []
        for cp in cps:
            cp.wait()
        cps = []
        for g in range(hw // chunk):
            cps.append(pltpu.async_copy(
                gi_tab.at[hist_v.at[pl.ds(g * chunk, chunk)]],
                rows_v.at[pl.ds(g * chunk, chunk)], sem))
            if len(cps) >= wave:
                for cp in cps:
                    cp.wait()
                cps = []
        for cp in cps:
            cp.wait()
        pltpu.sync_copy(rows_v, gih_out.at[pl.ds(hb, hw)])
        pltpu.sync_copy(gu_v, gu_out.at[pl.ds(base, bw)])
        pltpu.sync_copy(gi_v, gi_out.at[pl.ds(base, bw)])

    return kern


def _tc_mlp(B, H, K):
    def body(gih2_r, gu_r, gi_r, wbd_r, wg_r, tb1_r, rw2_r,
             w1at_r, w1bt_r, w2t_r, b1_r, b2_r, gu_star_r, xui_r):
        dot = functools.partial(
            lax.dot_general,
            dimension_numbers=(((1,), (0,)), ((), ())),
            precision=lax.Precision.HIGHEST,
            preferred_element_type=jnp.float32)
        gu = gu_r[...]
        h2 = dot(gih2_r[...], wbd_r[...]) + dot(gu, wg_r[...]) + tb1_r[...]
        h2 = jnp.where(h2 >= 0, h2, 0.01 * h2)
        gu_star = dot(h2, rw2_r[...]) + b2_r[...]
        hp = dot(gu, w1at_r[...]) + dot(gi_r[...], w1bt_r[...]) + b1_r[...]
        hp = jnp.where(hp >= 0, hp, 0.01 * hp)
        gui = dot(hp, w2t_r[...]) + b2_r[...]
        gu_star_r[...] = gu_star
        xui_r[...] = jnp.sum(gu_star * gui, axis=1, keepdims=True)

    return pl.pallas_call(
        body,
        out_shape=[
            jax.ShapeDtypeStruct((B, K), jnp.float32),
            jax.ShapeDtypeStruct((B, 1), jnp.float32),
        ],
    )


def kernel(users, items, Gu, Gi, W1, b1, W2, b2, ui):
    B = users.shape[0]
    NU, K = Gu.shape
    H = ui.shape[1] // NU

    ui_flat = ui.reshape(2 * NU * H)
    hpos = (((users.astype(jnp.int32) + NU) * H)[:, None]
            + jnp.arange(H, dtype=jnp.int32)[None, :]).reshape(B * H)
    gih, gu, gamma_i = _sc_gather(B, H, K, Gu.dtype)(
        users, items, hpos, ui_flat, Gu, Gi)
    gih2 = gih.reshape(B, H * K)

    w1at = W1[:, :K].T
    w1bt = W1[:, K:].T
    w2t = W2.T
    eye = jnp.eye(H, dtype=W1.dtype)
    wbd = jnp.kron(eye, w1bt)
    wg = jnp.tile(w1at, (1, H))
    tb1 = jnp.tile(b1, H)[None, :]
    rw2 = jnp.tile(w2t / H, (H, 1))
    gu_star, xui = _tc_mlp(B, H, K)(
        gih2, gu, gamma_i, wbd, wg, tb1, rw2,
        w1at, w1bt, w2t, b1[None, :], b2[None, :])
    return (xui.reshape(B), gu_star, gamma_i)


# speedup vs baseline: 1.0508x; 1.0508x over previous
import functools

import jax
import jax.numpy as jnp
from jax import lax
from jax.experimental import pallas as pl
from jax.experimental.pallas import tpu as pltpu
from jax.experimental.pallas import tpu_sc as plsc


def _sc_gather(B, H, K, dtype, chunk=128, wave=8):
    info = plsc.get_sparse_core_info()
    NC, NS = info.num_cores, info.num_subcores
    NW = NC * NS
    bw = B // NW
    hw = bw * H

    mesh = plsc.VectorSubcoreMesh(core_axis_name="c", subcore_axis_name="s")

    @functools.partial(
        pl.kernel,
        mesh=mesh,
        out_type=[
            jax.ShapeDtypeStruct((B * H, K), dtype),
            jax.ShapeDtypeStruct((B, K), dtype),
            jax.ShapeDtypeStruct((B, K), dtype),
        ],
        scratch_types=[
            pltpu.VMEM((bw,), jnp.int32),
            pltpu.VMEM((bw,), jnp.int32),
            pltpu.VMEM((hw,), jnp.int32),
            pltpu.VMEM((hw,), jnp.int32),
            pltpu.VMEM((bw, K), dtype),
            pltpu.VMEM((bw, K), dtype),
            pltpu.VMEM((hw, K), dtype),
            pltpu.SemaphoreType.DMA,
        ],
        compiler_params=pltpu.CompilerParams(use_tc_tiling_on_sc=False),
    )
    def kern(users_r, items_r, hpos_r, ui_r, gu_tab, gi_tab,
             gih_out, gu_out, gi_out,
             idx_u, idx_i, pos_v, hist_v, gu_v, gi_v, rows_v, sem):
        wid = lax.axis_index("s") * NC + lax.axis_index("c")
        base = wid * bw
        hb = wid * hw
        pltpu.sync_copy(users_r.at[pl.ds(base, bw)], idx_u)
        pltpu.sync_copy(items_r.at[pl.ds(base, bw)], idx_i)
        pltpu.sync_copy(hpos_r.at[pl.ds(hb, hw)], pos_v)
        cps = [pltpu.async_copy(gu_tab.at[idx_u], gu_v, sem),
               pltpu.async_copy(gi_tab.at[idx_i], gi_v, sem)]
        for g in range(hw // chunk):
            cps.append(pltpu.async_copy(
                ui_r.at[pos_v.at[pl.ds(g * chunk, chunk)]],
                hist_v.at[pl.ds(g * chunk, chunk)], sem))
            if len(cps) >= wave:
                for cp in cps:
                    cp.wait()
                cps = []
        for cp in cps:
            cp.wait()
        cps = []
        for g in range(hw // chunk):
            cps.append(pltpu.async_copy(
                gi_tab.at[hist_v.at[pl.ds(g * chunk, chunk)]],
                rows_v.at[pl.ds(g * chunk, chunk)], sem))
            if len(cps) >= wave:
                for cp in cps:
                    cp.wait()
                cps = []
        for cp in cps:
            cp.wait()
        pltpu.sync_copy(rows_v, gih_out.at[pl.ds(hb, hw)])
        pltpu.sync_copy(gu_v, gu_out.at[pl.ds(base, bw)])
        pltpu.sync_copy(gi_v, gi_out.at[pl.ds(base, bw)])

    return kern


def _tc_mlp(B, H, K):
    def body(gih2_r, gu_r, gi_r, wbd_r, wg_r, tb1_r, rw2_r,
             w1at_r, w1bt_r, w2t_r, b1_r, b2_r, gu_star_r, xui_r):
        dot = functools.partial(
            lax.dot_general,
            dimension_numbers=(((1,), (0,)), ((), ())),
            preferred_element_type=jnp.float32)
        gu = gu_r[...]
        h2 = dot(gih2_r[...], wbd_r[...]) + dot(gu, wg_r[...]) + tb1_r[...]
        h2 = jnp.where(h2 >= 0, h2, 0.01 * h2)
        gu_star = dot(h2, rw2_r[...]) + b2_r[...]
        hp = dot(gu, w1at_r[...]) + dot(gi_r[...], w1bt_r[...]) + b1_r[...]
        hp = jnp.where(hp >= 0, hp, 0.01 * hp)
        gui = dot(hp, w2t_r[...]) + b2_r[...]
        gu_star_r[...] = gu_star
        xui_r[...] = jnp.sum(gu_star * gui, axis=1, keepdims=True)

    return pl.pallas_call(
        body,
        out_shape=[
            jax.ShapeDtypeStruct((B, K), jnp.float32),
            jax.ShapeDtypeStruct((B, 1), jnp.float32),
        ],
    )


def kernel(users, items, Gu, Gi, W1, b1, W2, b2, ui):
    B = users.shape[0]
    NU, K = Gu.shape
    H = ui.shape[1] // NU

    ui_flat = ui.reshape(2 * NU * H)
    hpos = (((users.astype(jnp.int32) + NU) * H)[:, None]
            + jnp.arange(H, dtype=jnp.int32)[None, :]).reshape(B * H)
    gih, gu, gamma_i = _sc_gather(B, H, K, Gu.dtype)(
        users, items, hpos, ui_flat, Gu, Gi)
    gih2 = gih.reshape(B, H * K)

    w1at = W1[:, :K].T
    w1bt = W1[:, K:].T
    w2t = W2.T
    eye = jnp.eye(H, dtype=W1.dtype)
    wbd = jnp.kron(eye, w1bt)
    wg = jnp.tile(w1at, (1, H))
    tb1 = jnp.tile(b1, H)[None, :]
    rw2 = jnp.tile(w2t / H, (H, 1))
    gu_star, xui = _tc_mlp(B, H, K)(
        gih2, gu, gamma_i, wbd, wg, tb1, rw2,
        w1at, w1bt, w2t, b1[None, :], b2[None, :])
    return (xui.reshape(B), gu_star, gamma_i)
